# baseline (speedup 1.0000x reference)
import functools

import jax
import jax.numpy as jnp
import numpy as np
from jax import lax
from jax.experimental import pallas as pl
from jax.experimental.pallas import tpu as pltpu
from jax.experimental.pallas import tpu_sc as plsc

N = 10000
E = 320000
H = 128
C = 4096
BN_EPS = 1e-5

NC = 2
NS = 16
NW = NC * NS
K = 128
NCH = 80
PER_W = NCH * K
EP = PER_W * NW
NPAD = NS * 632
PADR = NPAD - N
RPT = NPAD // NS
RD = 400
RC = 1024


def _sc_scatter_partials(h, src_p, dst_p, zeros):
    mesh = plsc.VectorSubcoreMesh(core_axis_name="c", subcore_axis_name="s")

    @functools.partial(
        pl.kernel,
        mesh=mesh,
        out_type=jax.ShapeDtypeStruct((NC, NPAD, H), jnp.float32),
        scratch_types=[
            pltpu.VMEM((K,), jnp.int32),
            pltpu.VMEM((K,), jnp.int32),
            pltpu.VMEM((NCH, K), jnp.int32),
            pltpu.VMEM((K, H), jnp.float32),
            pltpu.VMEM((K, H), jnp.float32),
            pltpu.VMEM_SHARED((NPAD, H), jnp.float32),
            pltpu.SemaphoreType.DMA,
            pltpu.SemaphoreType.DMA,
            pltpu.SemaphoreType.DMA,
            pltpu.SemaphoreType.DMA,
        ],
    )
    def scatter_kernel(h_hbm, src_hbm, dst_hbm, zero_hbm, out_hbm,
                       sidx_a, sidx_b, dst_v, rows_a, rows_b, agg_sh,
                       sem_ga, sem_gb, sem_sa, sem_sb):
        cid = lax.axis_index("c")
        sid = lax.axis_index("s")
        wid = sid * NC + cid
        pltpu.sync_copy(dst_hbm.at[wid], dst_v)
        pltpu.sync_copy(zero_hbm.at[pl.ds(sid * RPT, RPT)],
                        agg_sh.at[pl.ds(sid * RPT, RPT)])
        plsc.subcore_barrier()

        base0 = wid * PER_W
        pltpu.sync_copy(src_hbm.at[pl.ds(base0, K)], sidx_a)
        pltpu.async_copy(h_hbm.at[sidx_a], rows_a, sem_ga)
        pltpu.sync_copy(src_hbm.at[pl.ds(base0 + K, K)], sidx_b)
        pltpu.async_copy(h_hbm.at[sidx_b], rows_b, sem_gb)

        @pl.loop(0, NCH, step=2)
        def _(c):
            base = wid * PER_W + c * K
            pltpu.make_async_copy(h_hbm.at[sidx_a], rows_a, sem_ga).wait()
            pltpu.async_copy(rows_a, agg_sh.at[dst_v.at[c]], sem_sa, add=True)
            pltpu.make_async_copy(h_hbm.at[sidx_b], rows_b, sem_gb).wait()
            pltpu.async_copy(rows_b, agg_sh.at[dst_v.at[c + 1]], sem_sb, add=True)

            @pl.when(c + 2 < NCH)
            def _():
                pltpu.make_async_copy(
                    rows_a, agg_sh.at[dst_v.at[c]], sem_sa).wait()
                pltpu.sync_copy(src_hbm.at[pl.ds(base + 2 * K, K)], sidx_a)
                pltpu.async_copy(h_hbm.at[sidx_a], rows_a, sem_ga)
                pltpu.make_async_copy(
                    rows_b, agg_sh.at[dst_v.at[c + 1]], sem_sb).wait()
                pltpu.sync_copy(src_hbm.at[pl.ds(base + 3 * K, K)], sidx_b)
                pltpu.async_copy(h_hbm.at[sidx_b], rows_b, sem_gb)

        pltpu.make_async_copy(
            rows_a, agg_sh.at[dst_v.at[NCH - 2]], sem_sa).wait()
        pltpu.make_async_copy(
            rows_b, agg_sh.at[dst_v.at[NCH - 1]], sem_sb).wait()
        plsc.subcore_barrier()
        pltpu.sync_copy(agg_sh.at[pl.ds(sid * RPT, RPT)],
                        out_hbm.at[cid, pl.ds(sid * RPT, RPT)])

    return scatter_kernel(h, src_p, dst_p, zeros)


def _sc_gather_pairs(h, ca, cb, first_edge):
    mesh = plsc.VectorSubcoreMesh(core_axis_name="c", subcore_axis_name="s")
    ck = C // NW

    @functools.partial(
        pl.kernel,
        mesh=mesh,
        out_type=(
            jax.ShapeDtypeStruct((C, H), jnp.float32),
            jax.ShapeDtypeStruct((C, H), jnp.float32),
            jax.ShapeDtypeStruct((2, H), jnp.float32),
        ),
        scratch_types=[
            pltpu.VMEM((ck,), jnp.int32),
            pltpu.VMEM((ck, H), jnp.float32),
            pltpu.VMEM((2,), jnp.int32),
            pltpu.VMEM((2, H), jnp.float32),
            pltpu.SemaphoreType.DMA,
        ],
    )
    def gather_kernel(h_hbm, ca_hbm, cb_hbm, fe_hbm, oa, ob, ouv,
                      idx_v, rows_v, fe_v, fr_v, sem):
        cid = lax.axis_index("c")
        sid = lax.axis_index("s")
        wid = sid * NC + cid
        base = wid * ck
        pltpu.sync_copy(ca_hbm.at[pl.ds(base, ck)], idx_v)
        pltpu.async_copy(h_hbm.at[idx_v], rows_v, sem).wait()
        pltpu.sync_copy(rows_v, oa.at[pl.ds(base, ck)])
        pltpu.sync_copy(cb_hbm.at[pl.ds(base, ck)], idx_v)
        pltpu.async_copy(h_hbm.at[idx_v], rows_v, sem).wait()
        pltpu.sync_copy(rows_v, ob.at[pl.ds(base, ck)])

        @pl.when(wid == 0)
        def _():
            pltpu.sync_copy(fe_hbm, fe_v)
            pltpu.async_copy(h_hbm.at[fe_v], fr_v, sem).wait()
            pltpu.sync_copy(fr_v, ouv)

    return gather_kernel(h, ca, cb, first_edge)


def _dense_body(h_ref, p_ref, w1_ref, w2_ref, eps_ref, b1_ref, b2_ref,
                sc_ref, be_ref, o_ref):
    h = h_ref[...]
    z0 = eps_ref[...] * h + p_ref[0] + p_ref[1]
    z1 = jnp.maximum(
        jnp.dot(z0, w1_ref[...], preferred_element_type=jnp.float32)
        + b1_ref[...], 0.0)
    z2 = (jnp.dot(z1, w2_ref[...], preferred_element_type=jnp.float32)
          + b2_ref[...])
    o_ref[...] = z2 * sc_ref[...] + be_ref[...] + h


def _tc_dense(h, parts, lp):
    eps1 = jnp.broadcast_to((1.0 + lp['eps'])[None, None], (1, H))
    scale = (lp['gamma'] * np.float32(1.0 / np.sqrt(1.0 + BN_EPS))).reshape(1, H)
    row = pl.BlockSpec((1, H), lambda i: (0, 0))
    return pl.pallas_call(
        _dense_body,
        grid=(N // RD,),
        in_specs=[
            pl.BlockSpec((RD, H), lambda i: (i, 0)),
            pl.BlockSpec((2, RD, H), lambda i: (0, i, 0)),
            pl.BlockSpec((H, H), lambda i: (0, 0)),
            pl.BlockSpec((H, H), lambda i: (0, 0)),
            row, row, row, row, row,
        ],
        out_specs=pl.BlockSpec((RD, H), lambda i: (i, 0)),
        out_shape=jax.ShapeDtypeStruct((N, H), jnp.float32),
    )(h, parts, lp['W1'], lp['W2'], eps1, lp['b1'].reshape(1, H),
      lp['b2'].reshape(1, H), scale, lp['beta'].reshape(1, H))


def _head_body(ha_ref, hb_ref, huv_ref, wt2_ref, pt2_ref, tv_ref,
               ewus_ref, ewud_ref, ews_ref, ewd_ref, ewt_ref, eb1_ref,
               ew2_ref, eb2_ref,
               owus_ref, owud_ref, ows_ref, owd_ref, owt_ref, ob1_ref,
               ow2_ref, ob2_ref,
               op_ref, oo_ref):
    hu = huv_ref[0:1, :]
    hv = huv_ref[1:2, :]
    us = hu + hv
    ud = jnp.abs(hu - hv)
    tz = wt2_ref[...] * tv_ref[...] + pt2_ref[...]
    lane = lax.broadcasted_iota(jnp.int32, (1, H), 1)
    temb = jnp.where(lane < H // 2, jnp.sin(tz), jnp.cos(tz))

    def mm(a, w_ref):
        return jnp.dot(a, w_ref[...], preferred_element_type=jnp.float32)

    bias_e = eb1_ref[...] + mm(us, ewus_ref) + mm(ud, ewud_ref) + mm(temb, ewt_ref)
    bias_o = ob1_ref[...] + mm(us, owus_ref) + mm(ud, owud_ref) + mm(temb, owt_ref)

    a = ha_ref[...]
    b = hb_ref[...]
    s = a + b
    d = jnp.abs(a - b)
    ze = jnp.maximum(mm(s, ews_ref) + mm(d, ewd_ref) + bias_e, 0.0)
    op_ref[...] = mm(ze, ew2_ref) + eb2_ref[...]
    zo = jnp.maximum(mm(s, ows_ref) + mm(d, owd_ref) + bias_o, 0.0)
    oo_ref[...] = mm(zo, ow2_ref) + ob2_ref[...]


def _tc_head(ha, hb, huv, t, params):
    wtcol = params['Wt'][:, 0]
    wt2 = jnp.concatenate([wtcol, wtcol]).reshape(1, H)
    pt2 = jnp.concatenate([params['Pt'], params['Pt']]).reshape(1, H)
    tv = jnp.broadcast_to(jnp.float32(t)[None, None], (1, H))
    ew = params['ep_W1']
    ow = params['or_W1']
    full = pl.BlockSpec((H, H), lambda i: (0, 0))
    row = pl.BlockSpec((1, H), lambda i: (0, 0))
    outs = pl.pallas_call(
        _head_body,
        grid=(C // RC,),
        in_specs=[
            pl.BlockSpec((RC, H), lambda i: (i, 0)),
            pl.BlockSpec((RC, H), lambda i: (i, 0)),
            pl.BlockSpec((2, H), lambda i: (0, 0)),
            row, row, row,
            full, full, full, full, full, row,
            pl.BlockSpec((H, 1), lambda i: (0, 0)),
            pl.BlockSpec((1, 1), lambda i: (0, 0)),
            full, full, full, full, full, row,
            pl.BlockSpec((H, 2), lambda i: (0, 0)),
            pl.BlockSpec((1, 2), lambda i: (0, 0)),
        ],
        out_specs=[
            pl.BlockSpec((RC, 1), lambda i: (i, 0)),
            pl.BlockSpec((RC, 2), lambda i: (i, 0)),
        ],
        out_shape=[
            jax.ShapeDtypeStruct((C, 1), jnp.float32),
            jax.ShapeDtypeStruct((C, 2), jnp.float32),
        ],
    )(ha, hb, huv, wt2, pt2, tv,
      ew[0:H], ew[H:2 * H], ew[2 * H:3 * H], ew[3 * H:4 * H], ew[4 * H:5 * H],
      params['ep_b1'].reshape(1, H), params['ep_W2'],
      params['ep_b2'].reshape(1, 1),
      ow[0:H], ow[H:2 * H], ow[2 * H:3 * H], ow[3 * H:4 * H], ow[4 * H:5 * H],
      params['or_b1'].reshape(1, H), params['or_W2'],
      params['or_b2'].reshape(1, 2))
    return outs


def kernel(x, edge_index, first_edge, candidate_edges, t, params):
    src = edge_index[0]
    dst = edge_index[1]
    pad = EP - E
    ar = jnp.arange(pad, dtype=jnp.int32)
    src_p = jnp.concatenate([src, ar % N])
    dst_p = jnp.concatenate([dst, N + (ar % PADR)]).reshape(NW, NCH, K)
    zeros = jnp.zeros((NPAD, H), jnp.float32)

    h = x
    for lp in params['layers']:
        parts = _sc_scatter_partials(h, src_p, dst_p, zeros)
        h = _tc_dense(h, parts, lp)

    ca = candidate_edges[:, 0]
    cb = candidate_edges[:, 1]
    ha, hb, huv = _sc_gather_pairs(h, ca, cb, first_edge)
    partner, orient = _tc_head(ha, hb, huv, t, params)
    return partner[:, 0], orient


# speedup vs baseline: 1.2929x; 1.2929x over previous
import functools

import jax
import jax.numpy as jnp
import numpy as np
from jax import lax
from jax.experimental import pallas as pl
from jax.experimental.pallas import tpu as pltpu
from jax.experimental.pallas import tpu_sc as plsc

N = 10000
E = 320000
H = 128
C = 4096
BN_EPS = 1e-5

NC = 2
NS = 16
NW = NC * NS
K = 128
NCH = 80
PER_W = NCH * K
EP = PER_W * NW
NPAD = NS * 632
PADR = NPAD - N
RPT = NPAD // NS
RD = 1000
RC = 1024


def _sc_scatter_partials(h, src_p, dst_p, zeros):
    mesh = plsc.VectorSubcoreMesh(core_axis_name="c", subcore_axis_name="s")

    @functools.partial(
        pl.kernel,
        mesh=mesh,
        out_type=jax.ShapeDtypeStruct((NC, NPAD, H), jnp.float32),
        scratch_types=[
            pltpu.VMEM((K,), jnp.int32),
            pltpu.VMEM((K,), jnp.int32),
            pltpu.VMEM((NCH, K), jnp.int32),
            pltpu.VMEM((K, H), jnp.float32),
            pltpu.VMEM((K, H), jnp.float32),
            pltpu.VMEM_SHARED((NPAD, H), jnp.float32),
            pltpu.SemaphoreType.DMA,
            pltpu.SemaphoreType.DMA,
            pltpu.SemaphoreType.DMA,
            pltpu.SemaphoreType.DMA,
            pltpu.SemaphoreType.DMA,
        ],
    )
    def scatter_kernel(h_hbm, src_hbm, dst_hbm, zero_hbm, out_hbm,
                       sidx_a, sidx_b, dst_v, rows_a, rows_b, agg_sh,
                       sem_ga, sem_gb, sem_ia, sem_ib, sem_d):
        cid = lax.axis_index("c")
        sid = lax.axis_index("s")
        wid = sid * NC + cid
        pltpu.async_copy(dst_hbm.at[wid], dst_v, sem_d)
        base0 = wid * PER_W
        pltpu.sync_copy(src_hbm.at[pl.ds(base0, K)], sidx_a)
        pltpu.async_copy(h_hbm.at[sidx_a], rows_a, sem_ga)
        pltpu.sync_copy(src_hbm.at[pl.ds(base0 + K, K)], sidx_b)
        pltpu.async_copy(h_hbm.at[sidx_b], rows_b, sem_gb)
        pltpu.sync_copy(zero_hbm.at[pl.ds(sid * RPT, RPT)],
                        agg_sh.at[pl.ds(sid * RPT, RPT)])
        pltpu.make_async_copy(dst_hbm.at[wid], dst_v, sem_d).wait()
        plsc.subcore_barrier()

        @pl.loop(0, NCH, step=2)
        def _(c):
            base = wid * PER_W + c * K
            pltpu.make_async_copy(h_hbm.at[sidx_a], rows_a, sem_ga).wait()

            @pl.when(c + 2 < NCH)
            def _():
                pltpu.async_copy(src_hbm.at[pl.ds(base + 2 * K, K)],
                                 sidx_a, sem_ia)

            pltpu.sync_copy(rows_a, agg_sh.at[dst_v.at[c]], add=True)

            @pl.when(c + 2 < NCH)
            def _():
                pltpu.make_async_copy(src_hbm.at[pl.ds(base + 2 * K, K)],
                                      sidx_a, sem_ia).wait()
                pltpu.async_copy(h_hbm.at[sidx_a], rows_a, sem_ga)

            pltpu.make_async_copy(h_hbm.at[sidx_b], rows_b, sem_gb).wait()

            @pl.when(c + 3 < NCH)
            def _():
                pltpu.async_copy(src_hbm.at[pl.ds(base + 3 * K, K)],
                                 sidx_b, sem_ib)

            pltpu.sync_copy(rows_b, agg_sh.at[dst_v.at[c + 1]], add=True)

            @pl.when(c + 3 < NCH)
            def _():
                pltpu.make_async_copy(src_hbm.at[pl.ds(base + 3 * K, K)],
                                      sidx_b, sem_ib).wait()
                pltpu.async_copy(h_hbm.at[sidx_b], rows_b, sem_gb)

        plsc.subcore_barrier()
        pltpu.sync_copy(agg_sh.at[pl.ds(sid * RPT, RPT)],
                        out_hbm.at[cid, pl.ds(sid * RPT, RPT)])

    return scatter_kernel(h, src_p, dst_p, zeros)


def _sc_gather_pairs(h, ca, cb, first_edge):
    mesh = plsc.VectorSubcoreMesh(core_axis_name="c", subcore_axis_name="s")
    ck = C // NW

    @functools.partial(
        pl.kernel,
        mesh=mesh,
        out_type=(
            jax.ShapeDtypeStruct((C, H), jnp.float32),
            jax.ShapeDtypeStruct((C, H), jnp.float32),
            jax.ShapeDtypeStruct((2, H), jnp.float32),
        ),
        scratch_types=[
            pltpu.VMEM((ck,), jnp.int32),
            pltpu.VMEM((ck, H), jnp.float32),
            pltpu.VMEM((2,), jnp.int32),
            pltpu.VMEM((2, H), jnp.float32),
            pltpu.SemaphoreType.DMA,
        ],
    )
    def gather_kernel(h_hbm, ca_hbm, cb_hbm, fe_hbm, oa, ob, ouv,
                      idx_v, rows_v, fe_v, fr_v, sem):
        cid = lax.axis_index("c")
        sid = lax.axis_index("s")
        wid = sid * NC + cid
        base = wid * ck
        pltpu.sync_copy(ca_hbm.at[pl.ds(base, ck)], idx_v)
        pltpu.async_copy(h_hbm.at[idx_v], rows_v, sem).wait()
        pltpu.sync_copy(rows_v, oa.at[pl.ds(base, ck)])
        pltpu.sync_copy(cb_hbm.at[pl.ds(base, ck)], idx_v)
        pltpu.async_copy(h_hbm.at[idx_v], rows_v, sem).wait()
        pltpu.sync_copy(rows_v, ob.at[pl.ds(base, ck)])

        @pl.when(wid == 0)
        def _():
            pltpu.sync_copy(fe_hbm, fe_v)
            pltpu.async_copy(h_hbm.at[fe_v], fr_v, sem).wait()
            pltpu.sync_copy(fr_v, ouv)

    return gather_kernel(h, ca, cb, first_edge)


def _dense_body(h_ref, p_ref, w1_ref, w2_ref, eps_ref, b1_ref, b2_ref,
                sc_ref, be_ref, o_ref):
    h = h_ref[...]
    z0 = eps_ref[...] * h + p_ref[0] + p_ref[1]
    z1 = jnp.maximum(
        jnp.dot(z0, w1_ref[...], preferred_element_type=jnp.float32)
        + b1_ref[...], 0.0)
    z2 = (jnp.dot(z1, w2_ref[...], preferred_element_type=jnp.float32)
          + b2_ref[...])
    o_ref[...] = z2 * sc_ref[...] + be_ref[...] + h


def _tc_dense(h, parts, lp):
    eps1 = jnp.broadcast_to((1.0 + lp['eps'])[None, None], (1, H))
    scale = (lp['gamma'] * np.float32(1.0 / np.sqrt(1.0 + BN_EPS))).reshape(1, H)
    row = pl.BlockSpec((1, H), lambda i: (0, 0))
    return pl.pallas_call(
        _dense_body,
        grid=(N // RD,),
        in_specs=[
            pl.BlockSpec((RD, H), lambda i: (i, 0)),
            pl.BlockSpec((2, RD, H), lambda i: (0, i, 0)),
            pl.BlockSpec((H, H), lambda i: (0, 0)),
            pl.BlockSpec((H, H), lambda i: (0, 0)),
            row, row, row, row, row,
        ],
        out_specs=pl.BlockSpec((RD, H), lambda i: (i, 0)),
        out_shape=jax.ShapeDtypeStruct((N, H), jnp.float32),
    )(h, parts, lp['W1'], lp['W2'], eps1, lp['b1'].reshape(1, H),
      lp['b2'].reshape(1, H), scale, lp['beta'].reshape(1, H))


def _head_body(ha_ref, hb_ref, huv_ref, wt2_ref, pt2_ref, tv_ref,
               ewus_ref, ewud_ref, ews_ref, ewd_ref, ewt_ref, eb1_ref,
               ew2_ref, eb2_ref,
               owus_ref, owud_ref, ows_ref, owd_ref, owt_ref, ob1_ref,
               ow2_ref, ob2_ref,
               op_ref, oo_ref):
    hu = huv_ref[0:1, :]
    hv = huv_ref[1:2, :]
    us = hu + hv
    ud = jnp.abs(hu - hv)
    tz = wt2_ref[...] * tv_ref[...] + pt2_ref[...]
    lane = lax.broadcasted_iota(jnp.int32, (1, H), 1)
    temb = jnp.where(lane < H // 2, jnp.sin(tz), jnp.cos(tz))

    def mm(a, w_ref):
        return jnp.dot(a, w_ref[...], preferred_element_type=jnp.float32)

    bias_e = eb1_ref[...] + mm(us, ewus_ref) + mm(ud, ewud_ref) + mm(temb, ewt_ref)
    bias_o = ob1_ref[...] + mm(us, owus_ref) + mm(ud, owud_ref) + mm(temb, owt_ref)

    a = ha_ref[...]
    b = hb_ref[...]
    s = a + b
    d = jnp.abs(a - b)
    ze = jnp.maximum(mm(s, ews_ref) + mm(d, ewd_ref) + bias_e, 0.0)
    op_ref[...] = mm(ze, ew2_ref) + eb2_ref[...]
    zo = jnp.maximum(mm(s, ows_ref) + mm(d, owd_ref) + bias_o, 0.0)
    oo_ref[...] = mm(zo, ow2_ref) + ob2_ref[...]


def _tc_head(ha, hb, huv, t, params):
    wtcol = params['Wt'][:, 0]
    wt2 = jnp.concatenate([wtcol, wtcol]).reshape(1, H)
    pt2 = jnp.concatenate([params['Pt'], params['Pt']]).reshape(1, H)
    tv = jnp.broadcast_to(jnp.float32(t)[None, None], (1, H))
    ew = params['ep_W1']
    ow = params['or_W1']
    full = pl.BlockSpec((H, H), lambda i: (0, 0))
    row = pl.BlockSpec((1, H), lambda i: (0, 0))
    outs = pl.pallas_call(
        _head_body,
        grid=(C // RC,),
        in_specs=[
            pl.BlockSpec((RC, H), lambda i: (i, 0)),
            pl.BlockSpec((RC, H), lambda i: (i, 0)),
            pl.BlockSpec((2, H), lambda i: (0, 0)),
            row, row, row,
            full, full, full, full, full, row,
            pl.BlockSpec((H, 1), lambda i: (0, 0)),
            pl.BlockSpec((1, 1), lambda i: (0, 0)),
            full, full, full, full, full, row,
            pl.BlockSpec((H, 2), lambda i: (0, 0)),
            pl.BlockSpec((1, 2), lambda i: (0, 0)),
        ],
        out_specs=[
            pl.BlockSpec((RC, 1), lambda i: (i, 0)),
            pl.BlockSpec((RC, 2), lambda i: (i, 0)),
        ],
        out_shape=[
            jax.ShapeDtypeStruct((C, 1), jnp.float32),
            jax.ShapeDtypeStruct((C, 2), jnp.float32),
        ],
    )(ha, hb, huv, wt2, pt2, tv,
      ew[0:H], ew[H:2 * H], ew[2 * H:3 * H], ew[3 * H:4 * H], ew[4 * H:5 * H],
      params['ep_b1'].reshape(1, H), params['ep_W2'],
      params['ep_b2'].reshape(1, 1),
      ow[0:H], ow[H:2 * H], ow[2 * H:3 * H], ow[3 * H:4 * H], ow[4 * H:5 * H],
      params['or_b1'].reshape(1, H), params['or_W2'],
      params['or_b2'].reshape(1, 2))
    return outs


def kernel(x, edge_index, first_edge, candidate_edges, t, params):
    src = edge_index[0]
    dst = edge_index[1]
    pad = EP - E
    ar = jnp.arange(pad, dtype=jnp.int32)
    src_p = jnp.concatenate([src, ar % N])
    dst_p = jnp.concatenate([dst, N + (ar % PADR)]).reshape(NW, NCH, K)
    zeros = jnp.zeros((NPAD, H), jnp.float32)

    h = x
    for lp in params['layers']:
        parts = _sc_scatter_partials(h, src_p, dst_p, zeros)
        h = _tc_dense(h, parts, lp)

    ca = candidate_edges[:, 0]
    cb = candidate_edges[:, 1]
    ha, hb, huv = _sc_gather_pairs(h, ca, cb, first_edge)
    partner, orient = _tc_head(ha, hb, huv, t, params)
    return partner[:, 0], orient


# speedup vs baseline: 1.3251x; 1.0249x over previous
import functools

import jax
import jax.numpy as jnp
import numpy as np
from jax import lax
from jax.experimental import pallas as pl
from jax.experimental.pallas import tpu as pltpu
from jax.experimental.pallas import tpu_sc as plsc

N = 10000
E = 320000
H = 128
C = 4096
BN_EPS = 1e-5

NC = 2
NS = 16
NW = NC * NS
K = 128
NCHG = E // K
NCH = (NCHG // NW) & ~1
NEXTRA = NCHG - NCH * NW
NPAD = NS * 632
PADR = NPAD - N
RPT = NPAD // NS
RD = 1000
RC = 1024


def _sc_scatter_partials(h, edge_index, zeros):
    mesh = plsc.VectorSubcoreMesh(core_axis_name="c", subcore_axis_name="s")

    @functools.partial(
        pl.kernel,
        mesh=mesh,
        out_type=jax.ShapeDtypeStruct((NC, NPAD, H), jnp.float32),
        scratch_types=[
            pltpu.VMEM((2, K), jnp.int32),
            pltpu.VMEM((2, K), jnp.int32),
            pltpu.VMEM((K, H), jnp.float32),
            pltpu.VMEM((K, H), jnp.float32),
            pltpu.VMEM_SHARED((NPAD, H), jnp.float32),
            pltpu.SemaphoreType.DMA,
            pltpu.SemaphoreType.DMA,
            pltpu.SemaphoreType.DMA,
            pltpu.SemaphoreType.DMA,
        ],
    )
    def scatter_kernel(h_hbm, ei_hbm, zero_hbm, out_hbm,
                       sd_a, sd_b, rows_a, rows_b, agg_sh,
                       sem_ga, sem_gb, sem_ia, sem_ib):
        cid = lax.axis_index("c")
        sid = lax.axis_index("s")
        wid = sid * NC + cid
        pltpu.sync_copy(ei_hbm.at[:, pl.ds(wid * K, K)], sd_a)
        pltpu.async_copy(h_hbm.at[sd_a.at[0]], rows_a, sem_ga)
        pltpu.sync_copy(ei_hbm.at[:, pl.ds((wid + NW) * K, K)], sd_b)
        pltpu.async_copy(h_hbm.at[sd_b.at[0]], rows_b, sem_gb)
        pltpu.sync_copy(zero_hbm.at[pl.ds(sid * RPT, RPT)],
                        agg_sh.at[pl.ds(sid * RPT, RPT)])
        plsc.subcore_barrier()

        @pl.loop(0, NCH, step=2)
        def _(j):
            o2 = (wid + (j + 2) * NW) * K
            o3 = (wid + (j + 3) * NW) * K
            pltpu.make_async_copy(h_hbm.at[sd_a.at[0]], rows_a, sem_ga).wait()

            @pl.when(j + 2 < NCH)
            def _():
                pltpu.async_copy(ei_hbm.at[:, pl.ds(o2, K)], sd_a, sem_ia)

            pltpu.sync_copy(rows_a, agg_sh.at[sd_a.at[1]], add=True)

            @pl.when(j + 2 < NCH)
            def _():
                pltpu.make_async_copy(
                    ei_hbm.at[:, pl.ds(o2, K)], sd_a, sem_ia).wait()
                pltpu.async_copy(h_hbm.at[sd_a.at[0]], rows_a, sem_ga)

            pltpu.make_async_copy(h_hbm.at[sd_b.at[0]], rows_b, sem_gb).wait()

            @pl.when(j + 3 < NCH)
            def _():
                pltpu.async_copy(ei_hbm.at[:, pl.ds(o3, K)], sd_b, sem_ib)

            pltpu.sync_copy(rows_b, agg_sh.at[sd_b.at[1]], add=True)

            @pl.when(j + 3 < NCH)
            def _():
                pltpu.make_async_copy(
                    ei_hbm.at[:, pl.ds(o3, K)], sd_b, sem_ib).wait()
                pltpu.async_copy(h_hbm.at[sd_b.at[0]], rows_b, sem_gb)

        @pl.when(wid < NEXTRA)
        def _():
            ox = (NCH * NW + wid) * K
            pltpu.sync_copy(ei_hbm.at[:, pl.ds(ox, K)], sd_a)
            pltpu.async_copy(h_hbm.at[sd_a.at[0]], rows_a, sem_ga)
            pltpu.make_async_copy(h_hbm.at[sd_a.at[0]], rows_a, sem_ga).wait()
            pltpu.sync_copy(rows_a, agg_sh.at[sd_a.at[1]], add=True)

        plsc.subcore_barrier()
        pltpu.sync_copy(agg_sh.at[pl.ds(sid * RPT, RPT)],
                        out_hbm.at[cid, pl.ds(sid * RPT, RPT)])

    return scatter_kernel(h, edge_index, zeros)


def _sc_gather_pairs(h, ca, cb, first_edge):
    mesh = plsc.VectorSubcoreMesh(core_axis_name="c", subcore_axis_name="s")
    ck = C // NW

    @functools.partial(
        pl.kernel,
        mesh=mesh,
        out_type=(
            jax.ShapeDtypeStruct((C, H), jnp.float32),
            jax.ShapeDtypeStruct((C, H), jnp.float32),
            jax.ShapeDtypeStruct((2, H), jnp.float32),
        ),
        scratch_types=[
            pltpu.VMEM((ck,), jnp.int32),
            pltpu.VMEM((ck, H), jnp.float32),
            pltpu.VMEM((2,), jnp.int32),
            pltpu.VMEM((2, H), jnp.float32),
            pltpu.SemaphoreType.DMA,
        ],
    )
    def gather_kernel(h_hbm, ca_hbm, cb_hbm, fe_hbm, oa, ob, ouv,
                      idx_v, rows_v, fe_v, fr_v, sem):
        cid = lax.axis_index("c")
        sid = lax.axis_index("s")
        wid = sid * NC + cid
        base = wid * ck
        pltpu.sync_copy(ca_hbm.at[pl.ds(base, ck)], idx_v)
        pltpu.async_copy(h_hbm.at[idx_v], rows_v, sem).wait()
        pltpu.sync_copy(rows_v, oa.at[pl.ds(base, ck)])
        pltpu.sync_copy(cb_hbm.at[pl.ds(base, ck)], idx_v)
        pltpu.async_copy(h_hbm.at[idx_v], rows_v, sem).wait()
        pltpu.sync_copy(rows_v, ob.at[pl.ds(base, ck)])

        @pl.when(wid == 0)
        def _():
            pltpu.sync_copy(fe_hbm, fe_v)
            pltpu.async_copy(h_hbm.at[fe_v], fr_v, sem).wait()
            pltpu.sync_copy(fr_v, ouv)

    return gather_kernel(h, ca, cb, first_edge)


def _dense_body(h_ref, p_ref, w1_ref, w2_ref, eps_ref, b1_ref, b2_ref,
                sc_ref, be_ref, o_ref):
    h = h_ref[...]
    z0 = eps_ref[...] * h + p_ref[0] + p_ref[1]
    z1 = jnp.maximum(
        jnp.dot(z0, w1_ref[...], preferred_element_type=jnp.float32)
        + b1_ref[...], 0.0)
    z2 = (jnp.dot(z1, w2_ref[...], preferred_element_type=jnp.float32)
          + b2_ref[...])
    o_ref[...] = z2 * sc_ref[...] + be_ref[...] + h


def _tc_dense(h, parts, lp):
    eps1 = jnp.broadcast_to((1.0 + lp['eps'])[None, None], (1, H))
    scale = (lp['gamma'] * np.float32(1.0 / np.sqrt(1.0 + BN_EPS))).reshape(1, H)
    row = pl.BlockSpec((1, H), lambda i: (0, 0))
    return pl.pallas_call(
        _dense_body,
        grid=(N // RD,),
        in_specs=[
            pl.BlockSpec((RD, H), lambda i: (i, 0)),
            pl.BlockSpec((2, RD, H), lambda i: (0, i, 0)),
            pl.BlockSpec((H, H), lambda i: (0, 0)),
            pl.BlockSpec((H, H), lambda i: (0, 0)),
            row, row, row, row, row,
        ],
        out_specs=pl.BlockSpec((RD, H), lambda i: (i, 0)),
        out_shape=jax.ShapeDtypeStruct((N, H), jnp.float32),
    )(h, parts, lp['W1'], lp['W2'], eps1, lp['b1'].reshape(1, H),
      lp['b2'].reshape(1, H), scale, lp['beta'].reshape(1, H))


def _head_body(ha_ref, hb_ref, huv_ref, wt2_ref, pt2_ref, tv_ref,
               ewus_ref, ewud_ref, ews_ref, ewd_ref, ewt_ref, eb1_ref,
               ew2_ref, eb2_ref,
               owus_ref, owud_ref, ows_ref, owd_ref, owt_ref, ob1_ref,
               ow2_ref, ob2_ref,
               op_ref, oo_ref):
    hu = huv_ref[0:1, :]
    hv = huv_ref[1:2, :]
    us = hu + hv
    ud = jnp.abs(hu - hv)
    tz = wt2_ref[...] * tv_ref[...] + pt2_ref[...]
    lane = lax.broadcasted_iota(jnp.int32, (1, H), 1)
    temb = jnp.where(lane < H // 2, jnp.sin(tz), jnp.cos(tz))

    def mm(a, w_ref):
        return jnp.dot(a, w_ref[...], preferred_element_type=jnp.float32)

    bias_e = eb1_ref[...] + mm(us, ewus_ref) + mm(ud, ewud_ref) + mm(temb, ewt_ref)
    bias_o = ob1_ref[...] + mm(us, owus_ref) + mm(ud, owud_ref) + mm(temb, owt_ref)

    a = ha_ref[...]
    b = hb_ref[...]
    s = a + b
    d = jnp.abs(a - b)
    ze = jnp.maximum(mm(s, ews_ref) + mm(d, ewd_ref) + bias_e, 0.0)
    op_ref[...] = mm(ze, ew2_ref) + eb2_ref[...]
    zo = jnp.maximum(mm(s, ows_ref) + mm(d, owd_ref) + bias_o, 0.0)
    oo_ref[...] = mm(zo, ow2_ref) + ob2_ref[...]


def _tc_head(ha, hb, huv, t, params):
    wtcol = params['Wt'][:, 0]
    wt2 = jnp.concatenate([wtcol, wtcol]).reshape(1, H)
    pt2 = jnp.concatenate([params['Pt'], params['Pt']]).reshape(1, H)
    tv = jnp.broadcast_to(jnp.float32(t)[None, None], (1, H))
    ew = params['ep_W1']
    ow = params['or_W1']
    full = pl.BlockSpec((H, H), lambda i: (0, 0))
    row = pl.BlockSpec((1, H), lambda i: (0, 0))
    outs = pl.pallas_call(
        _head_body,
        grid=(C // RC,),
        in_specs=[
            pl.BlockSpec((RC, H), lambda i: (i, 0)),
            pl.BlockSpec((RC, H), lambda i: (i, 0)),
            pl.BlockSpec((2, H), lambda i: (0, 0)),
            row, row, row,
            full, full, full, full, full, row,
            pl.BlockSpec((H, 1), lambda i: (0, 0)),
            pl.BlockSpec((1, 1), lambda i: (0, 0)),
            full, full, full, full, full, row,
            pl.BlockSpec((H, 2), lambda i: (0, 0)),
            pl.BlockSpec((1, 2), lambda i: (0, 0)),
        ],
        out_specs=[
            pl.BlockSpec((RC, 1), lambda i: (i, 0)),
            pl.BlockSpec((RC, 2), lambda i: (i, 0)),
        ],
        out_shape=[
            jax.ShapeDtypeStruct((C, 1), jnp.float32),
            jax.ShapeDtypeStruct((C, 2), jnp.float32),
        ],
    )(ha, hb, huv, wt2, pt2, tv,
      ew[0:H], ew[H:2 * H], ew[2 * H:3 * H], ew[3 * H:4 * H], ew[4 * H:5 * H],
      params['ep_b1'].reshape(1, H), params['ep_W2'],
      params['ep_b2'].reshape(1, 1),
      ow[0:H], ow[H:2 * H], ow[2 * H:3 * H], ow[3 * H:4 * H], ow[4 * H:5 * H],
      params['or_b1'].reshape(1, H), params['or_W2'],
      params['or_b2'].reshape(1, 2))
    return outs


def kernel(x, edge_index, first_edge, candidate_edges, t, params):
    zeros = jnp.zeros((NPAD, H), jnp.float32)

    h = x
    for lp in params['layers']:
        parts = _sc_scatter_partials(h, edge_index, zeros)
        h = _tc_dense(h, parts, lp)

    ca = candidate_edges[:, 0]
    cb = candidate_edges[:, 1]
    ha, hb, huv = _sc_gather_pairs(h, ca, cb, first_edge)
    partner, orient = _tc_head(ha, hb, huv, t, params)
    return partner[:, 0], orient


# speedup vs baseline: 1.3276x; 1.0019x over previous
import functools

import jax
import jax.numpy as jnp
import numpy as np
from jax import lax
from jax.experimental import pallas as pl
from jax.experimental.pallas import tpu as pltpu
from jax.experimental.pallas import tpu_sc as plsc

N = 10000
E = 320000
H = 128
C = 4096
BN_EPS = 1e-5

NC = 2
NS = 16
NW = NC * NS
K = 128
NCHG = E // K
NCH = (NCHG // NW) & ~1
NEXTRA = NCHG - NCH * NW
NPAD = NS * 632
PADR = NPAD - N
RPT = NPAD // NS
RD = 1000
RC = 1024


def _sc_scatter_partials(h, edge_index, zeros):
    mesh = plsc.VectorSubcoreMesh(core_axis_name="c", subcore_axis_name="s")

    @functools.partial(
        pl.kernel,
        mesh=mesh,
        out_type=jax.ShapeDtypeStruct((NC, NPAD, H), jnp.float32),
        scratch_types=[
            pltpu.VMEM((2, K), jnp.int32),
            pltpu.VMEM((2, K), jnp.int32),
            pltpu.VMEM((K,), jnp.int32),
            pltpu.VMEM((K,), jnp.int32),
            pltpu.VMEM((K, H), jnp.float32),
            pltpu.VMEM((K, H), jnp.float32),
            pltpu.VMEM_SHARED((NPAD, H), jnp.float32),
            pltpu.SemaphoreType.DMA,
            pltpu.SemaphoreType.DMA,
            pltpu.SemaphoreType.DMA,
            pltpu.SemaphoreType.DMA,
        ],
    )
    def scatter_kernel(h_hbm, ei_hbm, zero_hbm, out_hbm,
                       sd_a, sd_b, didx_a, didx_b, rows_a, rows_b, agg_sh,
                       sem_ga, sem_gb, sem_ia, sem_ib):
        cid = lax.axis_index("c")
        sid = lax.axis_index("s")
        wid = sid * NC + cid
        pltpu.sync_copy(ei_hbm.at[:, pl.ds(wid * K, K)], sd_a)
        pltpu.async_copy(h_hbm.at[sd_a.at[0]], rows_a, sem_ga)
        pltpu.sync_copy(ei_hbm.at[:, pl.ds((wid + NW) * K, K)], sd_b)
        pltpu.async_copy(h_hbm.at[sd_b.at[0]], rows_b, sem_gb)
        pltpu.sync_copy(zero_hbm.at[pl.ds(sid * RPT, RPT)],
                        agg_sh.at[pl.ds(sid * RPT, RPT)])
        plsc.subcore_barrier()

        @pl.loop(0, NCH, step=2)
        def _(j):
            o2 = (wid + (j + 2) * NW) * K
            o3 = (wid + (j + 3) * NW) * K
            pltpu.make_async_copy(h_hbm.at[sd_a.at[0]], rows_a, sem_ga).wait()
            for i in range(K // 16):
                didx_a[pl.ds(i * 16, 16)] = sd_a[1, pl.ds(i * 16, 16)]

            @pl.when(j + 2 < NCH)
            def _():
                pltpu.async_copy(ei_hbm.at[:, pl.ds(o2, K)], sd_a, sem_ia)

            pltpu.sync_copy(rows_a, agg_sh.at[didx_a], add=True)

            @pl.when(j + 2 < NCH)
            def _():
                pltpu.make_async_copy(
                    ei_hbm.at[:, pl.ds(o2, K)], sd_a, sem_ia).wait()
                pltpu.async_copy(h_hbm.at[sd_a.at[0]], rows_a, sem_ga)

            pltpu.make_async_copy(h_hbm.at[sd_b.at[0]], rows_b, sem_gb).wait()
            for i in range(K // 16):
                didx_b[pl.ds(i * 16, 16)] = sd_b[1, pl.ds(i * 16, 16)]

            @pl.when(j + 3 < NCH)
            def _():
                pltpu.async_copy(ei_hbm.at[:, pl.ds(o3, K)], sd_b, sem_ib)

            pltpu.sync_copy(rows_b, agg_sh.at[didx_b], add=True)

            @pl.when(j + 3 < NCH)
            def _():
                pltpu.make_async_copy(
                    ei_hbm.at[:, pl.ds(o3, K)], sd_b, sem_ib).wait()
                pltpu.async_copy(h_hbm.at[sd_b.at[0]], rows_b, sem_gb)

        @pl.when(wid < NEXTRA)
        def _():
            ox = (NCH * NW + wid) * K
            pltpu.sync_copy(ei_hbm.at[:, pl.ds(ox, K)], sd_a)
            pltpu.async_copy(h_hbm.at[sd_a.at[0]], rows_a, sem_ga)
            pltpu.make_async_copy(h_hbm.at[sd_a.at[0]], rows_a, sem_ga).wait()
            pltpu.sync_copy(rows_a, agg_sh.at[sd_a.at[1]], add=True)

        plsc.subcore_barrier()
        pltpu.sync_copy(agg_sh.at[pl.ds(sid * RPT, RPT)],
                        out_hbm.at[cid, pl.ds(sid * RPT, RPT)])

    return scatter_kernel(h, edge_index, zeros)


def _sc_gather_pairs(h, ca, cb, first_edge):
    mesh = plsc.VectorSubcoreMesh(core_axis_name="c", subcore_axis_name="s")
    ck = C // NW

    @functools.partial(
        pl.kernel,
        mesh=mesh,
        out_type=(
            jax.ShapeDtypeStruct((C, H), jnp.float32),
            jax.ShapeDtypeStruct((C, H), jnp.float32),
            jax.ShapeDtypeStruct((2, H), jnp.float32),
        ),
        scratch_types=[
            pltpu.VMEM((ck,), jnp.int32),
            pltpu.VMEM((ck, H), jnp.float32),
            pltpu.VMEM((2,), jnp.int32),
            pltpu.VMEM((2, H), jnp.float32),
            pltpu.SemaphoreType.DMA,
        ],
    )
    def gather_kernel(h_hbm, ca_hbm, cb_hbm, fe_hbm, oa, ob, ouv,
                      idx_v, rows_v, fe_v, fr_v, sem):
        cid = lax.axis_index("c")
        sid = lax.axis_index("s")
        wid = sid * NC + cid
        base = wid * ck
        pltpu.sync_copy(ca_hbm.at[pl.ds(base, ck)], idx_v)
        pltpu.async_copy(h_hbm.at[idx_v], rows_v, sem).wait()
        pltpu.sync_copy(rows_v, oa.at[pl.ds(base, ck)])
        pltpu.sync_copy(cb_hbm.at[pl.ds(base, ck)], idx_v)
        pltpu.async_copy(h_hbm.at[idx_v], rows_v, sem).wait()
        pltpu.sync_copy(rows_v, ob.at[pl.ds(base, ck)])

        @pl.when(wid == 0)
        def _():
            pltpu.sync_copy(fe_hbm, fe_v)
            pltpu.async_copy(h_hbm.at[fe_v], fr_v, sem).wait()
            pltpu.sync_copy(fr_v, ouv)

    return gather_kernel(h, ca, cb, first_edge)


def _dense_body(h_ref, p_ref, w1_ref, w2_ref, eps_ref, b1_ref, b2_ref,
                sc_ref, be_ref, o_ref):
    h = h_ref[...]
    z0 = eps_ref[...] * h + p_ref[0] + p_ref[1]
    z1 = jnp.maximum(
        jnp.dot(z0, w1_ref[...], preferred_element_type=jnp.float32)
        + b1_ref[...], 0.0)
    z2 = (jnp.dot(z1, w2_ref[...], preferred_element_type=jnp.float32)
          + b2_ref[...])
    o_ref[...] = z2 * sc_ref[...] + be_ref[...] + h


def _tc_dense(h, parts, lp):
    eps1 = jnp.broadcast_to((1.0 + lp['eps'])[None, None], (1, H))
    scale = (lp['gamma'] * np.float32(1.0 / np.sqrt(1.0 + BN_EPS))).reshape(1, H)
    row = pl.BlockSpec((1, H), lambda i: (0, 0))
    return pl.pallas_call(
        _dense_body,
        grid=(N // RD,),
        in_specs=[
            pl.BlockSpec((RD, H), lambda i: (i, 0)),
            pl.BlockSpec((2, RD, H), lambda i: (0, i, 0)),
            pl.BlockSpec((H, H), lambda i: (0, 0)),
            pl.BlockSpec((H, H), lambda i: (0, 0)),
            row, row, row, row, row,
        ],
        out_specs=pl.BlockSpec((RD, H), lambda i: (i, 0)),
        out_shape=jax.ShapeDtypeStruct((N, H), jnp.float32),
    )(h, parts, lp['W1'], lp['W2'], eps1, lp['b1'].reshape(1, H),
      lp['b2'].reshape(1, H), scale, lp['beta'].reshape(1, H))


def _head_body(ha_ref, hb_ref, huv_ref, wt2_ref, pt2_ref, tv_ref,
               ewus_ref, ewud_ref, ews_ref, ewd_ref, ewt_ref, eb1_ref,
               ew2_ref, eb2_ref,
               owus_ref, owud_ref, ows_ref, owd_ref, owt_ref, ob1_ref,
               ow2_ref, ob2_ref,
               op_ref, oo_ref):
    hu = huv_ref[0:1, :]
    hv = huv_ref[1:2, :]
    us = hu + hv
    ud = jnp.abs(hu - hv)
    tz = wt2_ref[...] * tv_ref[...] + pt2_ref[...]
    lane = lax.broadcasted_iota(jnp.int32, (1, H), 1)
    temb = jnp.where(lane < H // 2, jnp.sin(tz), jnp.cos(tz))

    def mm(a, w_ref):
        return jnp.dot(a, w_ref[...], preferred_element_type=jnp.float32)

    bias_e = eb1_ref[...] + mm(us, ewus_ref) + mm(ud, ewud_ref) + mm(temb, ewt_ref)
    bias_o = ob1_ref[...] + mm(us, owus_ref) + mm(ud, owud_ref) + mm(temb, owt_ref)

    a = ha_ref[...]
    b = hb_ref[...]
    s = a + b
    d = jnp.abs(a - b)
    ze = jnp.maximum(mm(s, ews_ref) + mm(d, ewd_ref) + bias_e, 0.0)
    op_ref[...] = mm(ze, ew2_ref) + eb2_ref[...]
    zo = jnp.maximum(mm(s, ows_ref) + mm(d, owd_ref) + bias_o, 0.0)
    oo_ref[...] = mm(zo, ow2_ref) + ob2_ref[...]


def _tc_head(ha, hb, huv, t, params):
    wtcol = params['Wt'][:, 0]
    wt2 = jnp.concatenate([wtcol, wtcol]).reshape(1, H)
    pt2 = jnp.concatenate([params['Pt'], params['Pt']]).reshape(1, H)
    tv = jnp.broadcast_to(jnp.float32(t)[None, None], (1, H))
    ew = params['ep_W1']
    ow = params['or_W1']
    full = pl.BlockSpec((H, H), lambda i: (0, 0))
    row = pl.BlockSpec((1, H), lambda i: (0, 0))
    outs = pl.pallas_call(
        _head_body,
        grid=(C // RC,),
        in_specs=[
            pl.BlockSpec((RC, H), lambda i: (i, 0)),
            pl.BlockSpec((RC, H), lambda i: (i, 0)),
            pl.BlockSpec((2, H), lambda i: (0, 0)),
            row, row, row,
            full, full, full, full, full, row,
            pl.BlockSpec((H, 1), lambda i: (0, 0)),
            pl.BlockSpec((1, 1), lambda i: (0, 0)),
            full, full, full, full, full, row,
            pl.BlockSpec((H, 2), lambda i: (0, 0)),
            pl.BlockSpec((1, 2), lambda i: (0, 0)),
        ],
        out_specs=[
            pl.BlockSpec((RC, 1), lambda i: (i, 0)),
            pl.BlockSpec((RC, 2), lambda i: (i, 0)),
        ],
        out_shape=[
            jax.ShapeDtypeStruct((C, 1), jnp.float32),
            jax.ShapeDtypeStruct((C, 2), jnp.float32),
        ],
    )(ha, hb, huv, wt2, pt2, tv,
      ew[0:H], ew[H:2 * H], ew[2 * H:3 * H], ew[3 * H:4 * H], ew[4 * H:5 * H],
      params['ep_b1'].reshape(1, H), params['ep_W2'],
      params['ep_b2'].reshape(1, 1),
      ow[0:H], ow[H:2 * H], ow[2 * H:3 * H], ow[3 * H:4 * H], ow[4 * H:5 * H],
      params['or_b1'].reshape(1, H), params['or_W2'],
      params['or_b2'].reshape(1, 2))
    return outs


def kernel(x, edge_index, first_edge, candidate_edges, t, params):
    zeros = jnp.zeros((NPAD, H), jnp.float32)

    h = x
    for lp in params['layers']:
        parts = _sc_scatter_partials(h, edge_index, zeros)
        h = _tc_dense(h, parts, lp)

    ca = candidate_edges[:, 0]
    cb = candidate_edges[:, 1]
    ha, hb, huv = _sc_gather_pairs(h, ca, cb, first_edge)
    partner, orient = _tc_head(ha, hb, huv, t, params)
    return partner[:, 0], orient


# speedup vs baseline: 1.3657x; 1.0287x over previous
import functools

import jax
import jax.numpy as jnp
import numpy as np
from jax import lax
from jax.experimental import pallas as pl
from jax.experimental.pallas import tpu as pltpu
from jax.experimental.pallas import tpu_sc as plsc

N = 10000
E = 320000
H = 128
C = 4096
BN_EPS = 1e-5

NC = 2
NS = 16
NW = NC * NS
K = 128
NCHG = E // K
NCH = (NCHG // NW) & ~1
NEXTRA = NCHG - NCH * NW
NPAD = NS * 632
PADR = NPAD - N
RPT = NPAD // NS
RD = 2000
RC = 1024


def _sc_scatter_partials(h, edge_index, zeros):
    mesh = plsc.VectorSubcoreMesh(core_axis_name="c", subcore_axis_name="s")

    @functools.partial(
        pl.kernel,
        mesh=mesh,
        out_type=jax.ShapeDtypeStruct((NC, NPAD, H), jnp.float32),
        scratch_types=[
            pltpu.VMEM((2, K), jnp.int32),
            pltpu.VMEM((2, K), jnp.int32),
            pltpu.VMEM((K,), jnp.int32),
            pltpu.VMEM((K,), jnp.int32),
            pltpu.VMEM((K, H), jnp.float32),
            pltpu.VMEM((K, H), jnp.float32),
            pltpu.VMEM_SHARED((NPAD, H), jnp.float32),
            pltpu.SemaphoreType.DMA,
            pltpu.SemaphoreType.DMA,
            pltpu.SemaphoreType.DMA,
            pltpu.SemaphoreType.DMA,
        ],
    )
    def scatter_kernel(h_hbm, ei_hbm, zero_hbm, out_hbm,
                       sd_a, sd_b, didx_a, didx_b, rows_a, rows_b, agg_sh,
                       sem_ga, sem_gb, sem_ia, sem_ib):
        cid = lax.axis_index("c")
        sid = lax.axis_index("s")
        wid = sid * NC + cid
        pltpu.sync_copy(ei_hbm.at[:, pl.ds(wid * K, K)], sd_a)
        pltpu.async_copy(h_hbm.at[sd_a.at[0]], rows_a, sem_ga)
        pltpu.sync_copy(ei_hbm.at[:, pl.ds((wid + NW) * K, K)], sd_b)
        pltpu.async_copy(h_hbm.at[sd_b.at[0]], rows_b, sem_gb)
        pltpu.sync_copy(zero_hbm.at[pl.ds(sid * RPT, RPT)],
                        agg_sh.at[pl.ds(sid * RPT, RPT)])
        plsc.subcore_barrier()

        @pl.loop(0, NCH, step=2)
        def _(j):
            o2 = (wid + (j + 2) * NW) * K
            o3 = (wid + (j + 3) * NW) * K
            pltpu.make_async_copy(h_hbm.at[sd_a.at[0]], rows_a, sem_ga).wait()
            for i in range(K // 16):
                didx_a[pl.ds(i * 16, 16)] = sd_a[1, pl.ds(i * 16, 16)]

            @pl.when(j + 2 < NCH)
            def _():
                pltpu.async_copy(ei_hbm.at[:, pl.ds(o2, K)], sd_a, sem_ia)

            pltpu.sync_copy(rows_a, agg_sh.at[didx_a], add=True)

            @pl.when(j + 2 < NCH)
            def _():
                pltpu.make_async_copy(
                    ei_hbm.at[:, pl.ds(o2, K)], sd_a, sem_ia).wait()
                pltpu.async_copy(h_hbm.at[sd_a.at[0]], rows_a, sem_ga)

            pltpu.make_async_copy(h_hbm.at[sd_b.at[0]], rows_b, sem_gb).wait()
            for i in range(K // 16):
                didx_b[pl.ds(i * 16, 16)] = sd_b[1, pl.ds(i * 16, 16)]

            @pl.when(j + 3 < NCH)
            def _():
                pltpu.async_copy(ei_hbm.at[:, pl.ds(o3, K)], sd_b, sem_ib)

            pltpu.sync_copy(rows_b, agg_sh.at[didx_b], add=True)

            @pl.when(j + 3 < NCH)
            def _():
                pltpu.make_async_copy(
                    ei_hbm.at[:, pl.ds(o3, K)], sd_b, sem_ib).wait()
                pltpu.async_copy(h_hbm.at[sd_b.at[0]], rows_b, sem_gb)

        @pl.when(wid < NEXTRA)
        def _():
            ox = (NCH * NW + wid) * K
            pltpu.sync_copy(ei_hbm.at[:, pl.ds(ox, K)], sd_a)
            pltpu.async_copy(h_hbm.at[sd_a.at[0]], rows_a, sem_ga)
            pltpu.make_async_copy(h_hbm.at[sd_a.at[0]], rows_a, sem_ga).wait()
            pltpu.sync_copy(rows_a, agg_sh.at[sd_a.at[1]], add=True)

        plsc.subcore_barrier()
        pltpu.sync_copy(agg_sh.at[pl.ds(sid * RPT, RPT)],
                        out_hbm.at[cid, pl.ds(sid * RPT, RPT)])

    return scatter_kernel(h, edge_index, zeros)


def _sc_gather_pairs(h, ca, cb, first_edge):
    mesh = plsc.VectorSubcoreMesh(core_axis_name="c", subcore_axis_name="s")
    ck = C // NW

    @functools.partial(
        pl.kernel,
        mesh=mesh,
        out_type=(
            jax.ShapeDtypeStruct((C, H), jnp.float32),
            jax.ShapeDtypeStruct((C, H), jnp.float32),
            jax.ShapeDtypeStruct((2, H), jnp.float32),
        ),
        scratch_types=[
            pltpu.VMEM((ck,), jnp.int32),
            pltpu.VMEM((ck, H), jnp.float32),
            pltpu.VMEM((2,), jnp.int32),
            pltpu.VMEM((2, H), jnp.float32),
            pltpu.SemaphoreType.DMA,
        ],
    )
    def gather_kernel(h_hbm, ca_hbm, cb_hbm, fe_hbm, oa, ob, ouv,
                      idx_v, rows_v, fe_v, fr_v, sem):
        cid = lax.axis_index("c")
        sid = lax.axis_index("s")
        wid = sid * NC + cid
        base = wid * ck
        pltpu.sync_copy(ca_hbm.at[pl.ds(base, ck)], idx_v)
        pltpu.async_copy(h_hbm.at[idx_v], rows_v, sem).wait()
        pltpu.sync_copy(rows_v, oa.at[pl.ds(base, ck)])
        pltpu.sync_copy(cb_hbm.at[pl.ds(base, ck)], idx_v)
        pltpu.async_copy(h_hbm.at[idx_v], rows_v, sem).wait()
        pltpu.sync_copy(rows_v, ob.at[pl.ds(base, ck)])

        @pl.when(wid == 0)
        def _():
            pltpu.sync_copy(fe_hbm, fe_v)
            pltpu.async_copy(h_hbm.at[fe_v], fr_v, sem).wait()
            pltpu.sync_copy(fr_v, ouv)

    return gather_kernel(h, ca, cb, first_edge)


def _dense_body(h_ref, p_ref, w1_ref, w2_ref, eps_ref, b1_ref, b2_ref,
                sc_ref, be_ref, o_ref):
    h = h_ref[...]
    z0 = eps_ref[...] * h + p_ref[0] + p_ref[1]
    z1 = jnp.maximum(
        jnp.dot(z0, w1_ref[...], preferred_element_type=jnp.float32)
        + b1_ref[...], 0.0)
    z2 = (jnp.dot(z1, w2_ref[...], preferred_element_type=jnp.float32)
          + b2_ref[...])
    o_ref[...] = z2 * sc_ref[...] + be_ref[...] + h


def _tc_dense(h, parts, lp):
    eps1 = jnp.broadcast_to((1.0 + lp['eps'])[None, None], (1, H))
    scale = (lp['gamma'] * np.float32(1.0 / np.sqrt(1.0 + BN_EPS))).reshape(1, H)
    row = pl.BlockSpec((1, H), lambda i: (0, 0))
    return pl.pallas_call(
        _dense_body,
        grid=(N // RD,),
        in_specs=[
            pl.BlockSpec((RD, H), lambda i: (i, 0)),
            pl.BlockSpec((2, RD, H), lambda i: (0, i, 0)),
            pl.BlockSpec((H, H), lambda i: (0, 0)),
            pl.BlockSpec((H, H), lambda i: (0, 0)),
            row, row, row, row, row,
        ],
        out_specs=pl.BlockSpec((RD, H), lambda i: (i, 0)),
        out_shape=jax.ShapeDtypeStruct((N, H), jnp.float32),
    )(h, parts, lp['W1'], lp['W2'], eps1, lp['b1'].reshape(1, H),
      lp['b2'].reshape(1, H), scale, lp['beta'].reshape(1, H))


def _head_body(ha_ref, hb_ref, huv_ref, wt2_ref, pt2_ref, tv_ref,
               ewus_ref, ewud_ref, ews_ref, ewd_ref, ewt_ref, eb1_ref,
               ew2_ref, eb2_ref,
               owus_ref, owud_ref, ows_ref, owd_ref, owt_ref, ob1_ref,
               ow2_ref, ob2_ref,
               op_ref, oo_ref):
    hu = huv_ref[0:1, :]
    hv = huv_ref[1:2, :]
    us = hu + hv
    ud = jnp.abs(hu - hv)
    tz = wt2_ref[...] * tv_ref[...] + pt2_ref[...]
    lane = lax.broadcasted_iota(jnp.int32, (1, H), 1)
    temb = jnp.where(lane < H // 2, jnp.sin(tz), jnp.cos(tz))

    def mm(a, w_ref):
        return jnp.dot(a, w_ref[...], preferred_element_type=jnp.float32)

    bias_e = eb1_ref[...] + mm(us, ewus_ref) + mm(ud, ewud_ref) + mm(temb, ewt_ref)
    bias_o = ob1_ref[...] + mm(us, owus_ref) + mm(ud, owud_ref) + mm(temb, owt_ref)

    a = ha_ref[...]
    b = hb_ref[...]
    s = a + b
    d = jnp.abs(a - b)
    ze = jnp.maximum(mm(s, ews_ref) + mm(d, ewd_ref) + bias_e, 0.0)
    op_ref[...] = (mm(ze, ew2_ref) + eb2_ref[...])[:, 0]
    zo = jnp.maximum(mm(s, ows_ref) + mm(d, owd_ref) + bias_o, 0.0)
    oo_ref[...] = mm(zo, ow2_ref) + ob2_ref[...]


def _tc_head(ha, hb, huv, t, params):
    wtcol = params['Wt'][:, 0]
    wt2 = jnp.concatenate([wtcol, wtcol]).reshape(1, H)
    pt2 = jnp.concatenate([params['Pt'], params['Pt']]).reshape(1, H)
    tv = jnp.broadcast_to(jnp.float32(t)[None, None], (1, H))
    ew = params['ep_W1']
    ow = params['or_W1']
    full = pl.BlockSpec((H, H), lambda i: (0, 0))
    row = pl.BlockSpec((1, H), lambda i: (0, 0))
    outs = pl.pallas_call(
        _head_body,
        grid=(C // RC,),
        in_specs=[
            pl.BlockSpec((RC, H), lambda i: (i, 0)),
            pl.BlockSpec((RC, H), lambda i: (i, 0)),
            pl.BlockSpec((2, H), lambda i: (0, 0)),
            row, row, row,
            full, full, full, full, full, row,
            pl.BlockSpec((H, 1), lambda i: (0, 0)),
            pl.BlockSpec((1, 1), lambda i: (0, 0)),
            full, full, full, full, full, row,
            pl.BlockSpec((H, 2), lambda i: (0, 0)),
            pl.BlockSpec((1, 2), lambda i: (0, 0)),
        ],
        out_specs=[
            pl.BlockSpec((RC,), lambda i: (i,)),
            pl.BlockSpec((RC, 2), lambda i: (i, 0)),
        ],
        out_shape=[
            jax.ShapeDtypeStruct((C,), jnp.float32),
            jax.ShapeDtypeStruct((C, 2), jnp.float32),
        ],
    )(ha, hb, huv, wt2, pt2, tv,
      ew[0:H], ew[H:2 * H], ew[2 * H:3 * H], ew[3 * H:4 * H], ew[4 * H:5 * H],
      params['ep_b1'].reshape(1, H), params['ep_W2'],
      params['ep_b2'].reshape(1, 1),
      ow[0:H], ow[H:2 * H], ow[2 * H:3 * H], ow[3 * H:4 * H], ow[4 * H:5 * H],
      params['or_b1'].reshape(1, H), params['or_W2'],
      params['or_b2'].reshape(1, 2))
    return outs


def kernel(x, edge_index, first_edge, candidate_edges, t, params):
    zeros = jnp.zeros((NPAD, H), jnp.float32)

    h = x
    for lp in params['layers']:
        parts = _sc_scatter_partials(h, edge_index, zeros)
        h = _tc_dense(h, parts, lp)

    ca = candidate_edges[:, 0]
    cb = candidate_edges[:, 1]
    ha, hb, huv = _sc_gather_pairs(h, ca, cb, first_edge)
    partner, orient = _tc_head(ha, hb, huv, t, params)
    return partner, orient
